# baseline (speedup 1.0000x reference)
import jax
import jax.numpy as jnp
from jax.experimental import pallas as pl
from jax.experimental.pallas import tpu as pltpu

DIM = 512
HID = 2048
E = 8
HB = 2


def _gate_weights(logits):
    T = logits.shape[0]
    col = jax.lax.broadcasted_iota(jnp.int32, (T, E), 1)
    m1 = jnp.max(logits, axis=1, keepdims=True)
    big = jnp.int32(E)
    idx1 = jnp.min(jnp.where(logits == m1, col, big), axis=1, keepdims=True)
    masked = jnp.where(col == idx1, -jnp.inf, logits)
    m2 = jnp.max(masked, axis=1, keepdims=True)
    idx2 = jnp.min(jnp.where(masked == m2, col, big), axis=1, keepdims=True)
    e2 = jnp.exp(m2 - m1)
    p1 = 1.0 / (1.0 + e2)
    p2 = 1.0 - p1
    return jnp.where(col == idx1, p1, jnp.where(col == idx2, p2, 0.0))


def _moe_body(x_ref, gw_ref, w1_ref, w2_ref, o_ref, wf_ref):
    e = pl.program_id(0)
    xb = x_ref[...]

    @pl.when(e == 0)
    def _():
        logits = jax.lax.dot_general(
            xb, gw_ref[...], (((1,), (1,)), ((), ())),
            preferred_element_type=jnp.float32)
        wf_ref[...] = _gate_weights(logits)
        o_ref[...] = jnp.zeros_like(o_ref)

    w_full = wf_ref[...]
    col = jax.lax.broadcasted_iota(jnp.int32, w_full.shape, 1)
    we = jnp.sum(jnp.where(col == e, w_full, 0.0), axis=1, keepdims=True)
    hh = jax.lax.dot_general(
        xb.astype(jnp.bfloat16), w1_ref[0].astype(jnp.bfloat16),
        (((1,), (1,)), ((), ())),
        preferred_element_type=jnp.float32)
    hh = jnp.maximum(hh, 0.0).astype(jnp.bfloat16)
    y = jax.lax.dot_general(
        hh, w2_ref[0].astype(jnp.bfloat16), (((1,), (1,)), ((), ())),
        preferred_element_type=jnp.float32)
    o_ref[...] += we * y


@jax.jit
def kernel(x, gate_w, w1, w2):
    B, N, D = x.shape
    T = B * N
    out = pl.pallas_call(
        _moe_body,
        grid=(E,),
        in_specs=[
            pl.BlockSpec((T, D), lambda e: (0, 0)),
            pl.BlockSpec((E, D), lambda e: (0, 0)),
            pl.BlockSpec((1, HID, D), lambda e: (e, 0, 0)),
            pl.BlockSpec((1, D, HID), lambda e: (e, 0, 0)),
        ],
        out_specs=pl.BlockSpec((T, D), lambda e: (0, 0)),
        out_shape=jax.ShapeDtypeStruct((T, D), jnp.float32),
        scratch_shapes=[pltpu.VMEM((T, E), jnp.float32)],
    )(x.reshape(T, D), gate_w, w1, w2)
    return out.reshape(B, N, D)


# speedup vs baseline: 1.0939x; 1.0939x over previous
import jax
import jax.numpy as jnp
from jax.experimental import pallas as pl
from jax.experimental.pallas import tpu as pltpu

DIM = 512
HID = 2048
E = 8
NBUF = 3


def _gate_weights(logits):
    T = logits.shape[0]
    col = jax.lax.broadcasted_iota(jnp.int32, (T, E), 1)
    m1 = jnp.max(logits, axis=1, keepdims=True)
    big = jnp.int32(E)
    idx1 = jnp.min(jnp.where(logits == m1, col, big), axis=1, keepdims=True)
    masked = jnp.where(col == idx1, -jnp.inf, logits)
    m2 = jnp.max(masked, axis=1, keepdims=True)
    idx2 = jnp.min(jnp.where(masked == m2, col, big), axis=1, keepdims=True)
    e2 = jnp.exp(m2 - m1)
    p1 = 1.0 / (1.0 + e2)
    p2 = 1.0 - p1
    return jnp.where(col == idx1, p1, jnp.where(col == idx2, p2, 0.0))


def _moe_body(x_ref, gw_ref, w1_hbm, w2_hbm, o_ref, w1buf, w2buf, sems):
    def copies(e, b):
        return (
            pltpu.make_async_copy(w1_hbm.at[e], w1buf.at[b], sems.at[b, 0]),
            pltpu.make_async_copy(w2_hbm.at[e], w2buf.at[b], sems.at[b, 1]),
        )

    for e in range(NBUF):
        for c in copies(e, e):
            c.start()

    xb = x_ref[...]
    logits = jax.lax.dot_general(
        xb, gw_ref[...], (((1,), (1,)), ((), ())),
        preferred_element_type=jnp.float32)
    wf = _gate_weights(logits)

    for e in range(E):
        b = e % NBUF
        for c in copies(e, b):
            c.wait()
        hh = jax.lax.dot_general(
            xb, w1buf[b], (((1,), (1,)), ((), ())),
            preferred_element_type=jnp.float32)
        hh = jnp.maximum(hh, 0.0)
        y = jax.lax.dot_general(
            hh, w2buf[b], (((1,), (1,)), ((), ())),
            preferred_element_type=jnp.float32)
        contrib = wf[:, e:e + 1] * y
        if e == 0:
            o_ref[...] = contrib
        else:
            o_ref[...] += contrib
        if e + NBUF < E:
            for c in copies(e + NBUF, b):
                c.start()


@jax.jit
def kernel(x, gate_w, w1, w2):
    B, N, D = x.shape
    T = B * N
    out = pl.pallas_call(
        _moe_body,
        in_specs=[
            pl.BlockSpec(memory_space=pltpu.VMEM),
            pl.BlockSpec(memory_space=pltpu.VMEM),
            pl.BlockSpec(memory_space=pl.ANY),
            pl.BlockSpec(memory_space=pl.ANY),
        ],
        out_specs=pl.BlockSpec(memory_space=pltpu.VMEM),
        out_shape=jax.ShapeDtypeStruct((T, D), jnp.float32),
        scratch_shapes=[
            pltpu.VMEM((NBUF, HID, DIM), jnp.float32),
            pltpu.VMEM((NBUF, DIM, HID), jnp.float32),
            pltpu.SemaphoreType.DMA((NBUF, 2)),
        ],
    )(x.reshape(T, D), gate_w, w1, w2)
    return out.reshape(B, N, D)
